# baseline (speedup 1.0000x reference)
import functools

import jax
import jax.numpy as jnp
from jax import lax
from jax.experimental import pallas as pl
from jax.experimental.pallas import tpu as pltpu
from jax.experimental.pallas import tpu_sc as plsc

B = 16384
D = 64
NEGK = 5
C = 128
LANES = 16
UNROLL = 4


def _sc_scores(pos_u, pos_v, neg_flat, wu, wv):
    info = plsc.get_sparse_core_info()
    nw = info.num_cores * info.num_subcores
    epw = B // nw
    nchunk = epw // C
    mesh = plsc.VectorSubcoreMesh(core_axis_name="c", subcore_axis_name="s")

    buf = lambda shape, dt: [pltpu.VMEM(shape, dt) for _ in range(2)]

    @functools.partial(
        pl.kernel,
        out_type=[jax.ShapeDtypeStruct((B,), jnp.float32),
                  jax.ShapeDtypeStruct((B * NEGK,), jnp.float32)],
        mesh=mesh,
        scratch_types=[
            buf((C,), jnp.int32),
            buf((C,), jnp.int32),
            buf((C * NEGK,), jnp.int32),
            buf((C, D), jnp.float32),
            buf((C, D), jnp.float32),
            buf((C * NEGK, D), jnp.float32),
            pltpu.VMEM((C,), jnp.float32),
            pltpu.VMEM((C * NEGK,), jnp.float32),
            [pltpu.SemaphoreType.DMA for _ in range(2)],
        ],
        compiler_params=pltpu.CompilerParams(needs_layout_passes=False,
                                             use_tc_tiling_on_sc=False),
    )
    def scores(pos_u_hbm, pos_v_hbm, neg_hbm, wu_hbm, wv_hbm,
               pos_out, neg_out, iu, iv, ineg, ru, rv, rn, sp, sn, sem):
        wid = lax.axis_index("s") * info.num_cores + lax.axis_index("c")
        lane = jnp.arange(LANES, dtype=jnp.int32)

        def stage(c):
            p = c % 2
            b0 = wid * epw + c * C
            pltpu.sync_copy(pos_u_hbm.at[pl.ds(b0, C)], iu[p])
            pltpu.sync_copy(pos_v_hbm.at[pl.ds(b0, C)], iv[p])
            pltpu.sync_copy(neg_hbm.at[pl.ds(b0 * NEGK, C * NEGK)], ineg[p])
            return [pltpu.async_copy(wu_hbm.at[iu[p]], ru[p], sem[p]),
                    pltpu.async_copy(wv_hbm.at[iv[p]], rv[p], sem[p]),
                    pltpu.async_copy(wv_hbm.at[ineg[p]], rn[p], sem[p])]

        cps = stage(0)
        for c in range(nchunk):
            p = c % 2
            for cp in cps:
                cp.wait()
            if c + 1 < nchunk:
                cps = stage(c + 1)
            b0 = wid * epw + c * C
            for g in range(C // LANES):
                s = pl.ds(g * LANES, LANES)
                rowu = lane + (g * LANES)
                rown = [rowu * NEGK + n for n in range(NEGK)]

                def dbody(j, accs, p=p, rowu=rowu, rown=rown):
                    out = list(accs)
                    for k in range(UNROLL):
                        dcol = jnp.full((LANES,), j * UNROLL + k, jnp.int32)
                        xu = plsc.load_gather(ru[p], [rowu, dcol])
                        xv = plsc.load_gather(rv[p], [rowu, dcol])
                        out[0] = out[0] + xu * xv
                        for n in range(NEGK):
                            xn = plsc.load_gather(rn[p], [rown[n], dcol])
                            out[1 + n] = out[1 + n] + xn * xu
                    return tuple(out)

                z = jnp.zeros((LANES,), jnp.float32)
                accs = lax.fori_loop(0, D // UNROLL, dbody,
                                     (z,) * (1 + NEGK))
                sp[s] = accs[0]
                for n in range(NEGK):
                    plsc.store_scatter(sn, [rown[n]], accs[1 + n])
            pltpu.sync_copy(sp, pos_out.at[pl.ds(b0, C)])
            pltpu.sync_copy(sn, neg_out.at[pl.ds(b0 * NEGK, C * NEGK)])

    return scores(pos_u, pos_v, neg_flat, wu, wv)


def _loss(pos_s, neg_s):
    pos2 = pos_s.reshape(B // 128, 128)
    neg2 = neg_s.reshape(B * NEGK // 128, 128)

    def body(p_ref, n_ref, o_ref):
        p = jnp.clip(p_ref[...], -6.0, 6.0)
        n = jnp.clip(n_ref[...], -6.0, 6.0)
        lp = jnp.log1p(jnp.exp(-p))
        ln = jnp.log1p(jnp.exp(n))
        o_ref[0, 0] = (jnp.sum(lp) + jnp.sum(ln)) * (1.0 / B)

    out = pl.pallas_call(
        body,
        out_shape=jax.ShapeDtypeStruct((1, 1), jnp.float32),
        out_specs=pl.BlockSpec(memory_space=pltpu.SMEM),
    )(pos2, neg2)
    return out[0, 0]


TBLK = 16384


def _tc_transpose(wT):
    V = wT.shape[1]

    def body(in_ref, out_ref):
        r = jax.lax.broadcasted_iota(jnp.int32, (D, D), 0)
        c = jax.lax.broadcasted_iota(jnp.int32, (D, D), 1)
        eye = (r == c).astype(jnp.float32)
        out_ref[...] = jax.lax.dot_general(
            in_ref[...], eye, (((0,), (0,)), ((), ())),
            preferred_element_type=jnp.float32)

    return pl.pallas_call(
        body,
        grid=(pl.cdiv(V, TBLK),),
        in_specs=[pl.BlockSpec((D, TBLK), lambda i: (0, i))],
        out_specs=pl.BlockSpec((TBLK, D), lambda i: (i, 0)),
        out_shape=jax.ShapeDtypeStruct((V, D), jnp.float32),
    )(wT)


def kernel(pos_u, pos_v, neg_v, snd_u_weight, snd_v_weight):
    wu_row = _tc_transpose(snd_u_weight.T)
    wv_row = _tc_transpose(snd_v_weight.T)
    pos_s, neg_s = _sc_scores(pos_u, pos_v, neg_v.reshape(-1),
                              wu_row, wv_row)
    return _loss(pos_s, neg_s)


# speedup vs baseline: 2.1728x; 2.1728x over previous
import functools

import jax
import jax.numpy as jnp
from jax import lax
from jax.experimental import pallas as pl
from jax.experimental.pallas import tpu as pltpu
from jax.experimental.pallas import tpu_sc as plsc

B = 16384
D = 64
NEGK = 5
C = 64
LANES = 16
UNROLL = 4
DW = 128


def _sc_scores(pos_u, pos_v, neg_flat, wu, wv):
    info = plsc.get_sparse_core_info()
    nw = info.num_cores * info.num_subcores
    epw = B // nw
    nchunk = epw // C
    mesh = plsc.VectorSubcoreMesh(core_axis_name="c", subcore_axis_name="s")

    buf = lambda shape, dt: [pltpu.VMEM(shape, dt) for _ in range(2)]

    @functools.partial(
        pl.kernel,
        out_type=[jax.ShapeDtypeStruct((B,), jnp.float32),
                  jax.ShapeDtypeStruct((B * NEGK,), jnp.float32)],
        mesh=mesh,
        scratch_types=[
            buf((C,), jnp.int32),
            buf((C,), jnp.int32),
            buf((C * NEGK,), jnp.int32),
            buf((C, DW), jnp.float32),
            buf((C, DW), jnp.float32),
            buf((C * NEGK, DW), jnp.float32),
            pltpu.VMEM((C,), jnp.float32),
            pltpu.VMEM((C * NEGK,), jnp.float32),
            [pltpu.SemaphoreType.DMA for _ in range(2)],
        ],
        compiler_params=pltpu.CompilerParams(needs_layout_passes=False,
                                             use_tc_tiling_on_sc=False),
    )
    def scores(pos_u_hbm, pos_v_hbm, neg_hbm, wu_hbm, wv_hbm,
               pos_out, neg_out, iu, iv, ineg, ru, rv, rn, sp, sn, sem):
        wid = lax.axis_index("s") * info.num_cores + lax.axis_index("c")
        lane = jnp.arange(LANES, dtype=jnp.int32)

        def stage(c):
            p = c % 2
            b0 = wid * epw + c * C
            pltpu.sync_copy(pos_u_hbm.at[pl.ds(b0, C)], iu[p])
            pltpu.sync_copy(pos_v_hbm.at[pl.ds(b0, C)], iv[p])
            pltpu.sync_copy(neg_hbm.at[pl.ds(b0 * NEGK, C * NEGK)], ineg[p])
            return [pltpu.async_copy(wu_hbm.at[iu[p]], ru[p], sem[p]),
                    pltpu.async_copy(wv_hbm.at[iv[p]], rv[p], sem[p]),
                    pltpu.async_copy(wv_hbm.at[ineg[p]], rn[p], sem[p])]

        cps = stage(0)
        for c in range(nchunk):
            p = c % 2
            for cp in cps:
                cp.wait()
            if c + 1 < nchunk:
                cps = stage(c + 1)
            b0 = wid * epw + c * C
            for g in range(C // LANES):
                s = pl.ds(g * LANES, LANES)
                rowu = lane + (g * LANES)
                rown = [rowu * NEGK + n for n in range(NEGK)]

                def dbody(j, accs, p=p, rowu=rowu, rown=rown):
                    out = list(accs)
                    for k in range(UNROLL):
                        dcol = jnp.full((LANES,), j * UNROLL + k, jnp.int32)
                        xu = plsc.load_gather(ru[p], [rowu, dcol])
                        xv = plsc.load_gather(rv[p], [rowu, dcol])
                        out[0] = out[0] + xu * xv
                        for n in range(NEGK):
                            xn = plsc.load_gather(rn[p], [rown[n], dcol])
                            out[1 + n] = out[1 + n] + xn * xu
                    return tuple(out)

                z = jnp.zeros((LANES,), jnp.float32)
                accs = lax.fori_loop(0, D // UNROLL, dbody,
                                     (z,) * (1 + NEGK))
                sp[s] = accs[0]
                for n in range(NEGK):
                    plsc.store_scatter(sn, [rown[n]], accs[1 + n])
            pltpu.sync_copy(sp, pos_out.at[pl.ds(b0, C)])
            pltpu.sync_copy(sn, neg_out.at[pl.ds(b0 * NEGK, C * NEGK)])

    return scores(pos_u, pos_v, neg_flat, wu, wv)


def _loss(pos_s, neg_s):
    pos2 = pos_s.reshape(B // 128, 128)
    neg2 = neg_s.reshape(B * NEGK // 128, 128)

    def body(p_ref, n_ref, o_ref):
        p = jnp.clip(p_ref[...], -6.0, 6.0)
        n = jnp.clip(n_ref[...], -6.0, 6.0)
        lp = jnp.log1p(jnp.exp(-p))
        ln = jnp.log1p(jnp.exp(n))
        o_ref[0, 0] = (jnp.sum(lp) + jnp.sum(ln)) * (1.0 / B)

    out = pl.pallas_call(
        body,
        out_shape=jax.ShapeDtypeStruct((1, 1), jnp.float32),
        out_specs=pl.BlockSpec(memory_space=pltpu.SMEM),
    )(pos2, neg2)
    return out[0, 0]


TBLK = 16384


def _tc_transpose(wT):
    V = wT.shape[1]

    def body(in_ref, out_ref):
        r = jax.lax.broadcasted_iota(jnp.int32, (D, 2 * D), 0)
        c = jax.lax.broadcasted_iota(jnp.int32, (D, 2 * D), 1)
        eye = (r == c).astype(jnp.float32)
        out_ref[...] = jax.lax.dot_general(
            in_ref[...], eye, (((0,), (0,)), ((), ())),
            preferred_element_type=jnp.float32)

    return pl.pallas_call(
        body,
        grid=(pl.cdiv(V, TBLK),),
        in_specs=[pl.BlockSpec((D, TBLK), lambda i: (0, i))],
        out_specs=pl.BlockSpec((TBLK, 2 * D), lambda i: (i, 0)),
        out_shape=jax.ShapeDtypeStruct((V, 2 * D), jnp.float32),
    )(wT)


def kernel(pos_u, pos_v, neg_v, snd_u_weight, snd_v_weight):
    wu_row = _tc_transpose(snd_u_weight.T)
    wv_row = _tc_transpose(snd_v_weight.T)
    pos_s, neg_s = _sc_scores(pos_u, pos_v, neg_v.reshape(-1),
                              wu_row, wv_row)
    return _loss(pos_s, neg_s)


# speedup vs baseline: 2.2185x; 1.0210x over previous
import functools

import jax
import jax.numpy as jnp
from jax import lax
from jax.experimental import pallas as pl
from jax.experimental.pallas import tpu as pltpu
from jax.experimental.pallas import tpu_sc as plsc

B = 16384
D = 64
NEGK = 5
C = 64
LANES = 16
UNROLL = 4
DW = 128


def _sc_scores(pos_u, pos_v, neg_flat, wu, wv):
    info = plsc.get_sparse_core_info()
    nw = info.num_cores * info.num_subcores
    epw = B // nw
    nchunk = epw // C
    mesh = plsc.VectorSubcoreMesh(core_axis_name="c", subcore_axis_name="s")

    buf = lambda shape, dt: [pltpu.VMEM(shape, dt) for _ in range(2)]

    @functools.partial(
        pl.kernel,
        out_type=[jax.ShapeDtypeStruct((B,), jnp.float32),
                  jax.ShapeDtypeStruct((B * NEGK,), jnp.float32)],
        mesh=mesh,
        scratch_types=[
            buf((C,), jnp.int32),
            buf((C,), jnp.int32),
            buf((C * NEGK,), jnp.int32),
            buf((C, DW), jnp.float32),
            buf((C, DW), jnp.float32),
            buf((C * NEGK, DW), jnp.float32),
            pltpu.VMEM((C,), jnp.float32),
            pltpu.VMEM((C * NEGK,), jnp.float32),
            [pltpu.SemaphoreType.DMA for _ in range(2)],
        ],
        compiler_params=pltpu.CompilerParams(needs_layout_passes=False,
                                             use_tc_tiling_on_sc=False),
    )
    def scores(pos_u_hbm, pos_v_hbm, neg_hbm, wu_hbm, wv_hbm,
               pos_out, neg_out, iu, iv, ineg, ru, rv, rn, sp, sn, sem):
        wid = lax.axis_index("s") * info.num_cores + lax.axis_index("c")
        lane = jnp.arange(LANES, dtype=jnp.int32)

        def stage(c):
            p = c % 2
            b0 = wid * epw + c * C
            pltpu.sync_copy(pos_u_hbm.at[pl.ds(b0, C)], iu[p])
            pltpu.sync_copy(pos_v_hbm.at[pl.ds(b0, C)], iv[p])
            pltpu.sync_copy(neg_hbm.at[pl.ds(b0 * NEGK, C * NEGK)], ineg[p])
            return [pltpu.async_copy(wu_hbm.at[iu[p]], ru[p], sem[p]),
                    pltpu.async_copy(wv_hbm.at[iv[p]], rv[p], sem[p]),
                    pltpu.async_copy(wv_hbm.at[ineg[p]], rn[p], sem[p])]

        cps = stage(0)
        for c in range(nchunk):
            p = c % 2
            for cp in cps:
                cp.wait()
            if c + 1 < nchunk:
                cps = stage(c + 1)
            b0 = wid * epw + c * C
            for g in range(C // LANES):
                s = pl.ds(g * LANES, LANES)
                rowu = lane + (g * LANES)
                rown = [rowu * NEGK + n for n in range(NEGK)]

                def dbody(j, accs, p=p, rowu=rowu, rown=rown):
                    out = list(accs)
                    for k in range(UNROLL):
                        dcol = jnp.full((LANES,), j * UNROLL + k, jnp.int32)
                        xu = plsc.load_gather(ru[p], [rowu, dcol])
                        xv = plsc.load_gather(rv[p], [rowu, dcol])
                        out[0] = out[0] + xu * xv
                        for n in range(NEGK):
                            xn = plsc.load_gather(rn[p], [rown[n], dcol])
                            out[1 + n] = out[1 + n] + xn * xu
                    return tuple(out)

                z = jnp.zeros((LANES,), jnp.float32)
                accs = lax.fori_loop(0, D // UNROLL, dbody,
                                     (z,) * (1 + NEGK))
                sp[s] = accs[0]
                for n in range(NEGK):
                    plsc.store_scatter(sn, [rown[n]], accs[1 + n])
            pltpu.sync_copy(sp, pos_out.at[pl.ds(b0, C)])
            pltpu.sync_copy(sn, neg_out.at[pl.ds(b0 * NEGK, C * NEGK)])

    return scores(pos_u, pos_v, neg_flat, wu, wv)


def _loss(pos_s, neg_s):
    pos2 = pos_s.reshape(B // 128, 128)
    neg2 = neg_s.reshape(B * NEGK // 128, 128)

    def body(p_ref, n_ref, o_ref):
        p = jnp.clip(p_ref[...], -6.0, 6.0)
        n = jnp.clip(n_ref[...], -6.0, 6.0)
        lp = jnp.log1p(jnp.exp(-p))
        ln = jnp.log1p(jnp.exp(n))
        o_ref[0, 0] = (jnp.sum(lp) + jnp.sum(ln)) * (1.0 / B)

    out = pl.pallas_call(
        body,
        out_shape=jax.ShapeDtypeStruct((1, 1), jnp.float32),
        out_specs=pl.BlockSpec(memory_space=pltpu.SMEM),
    )(pos2, neg2)
    return out[0, 0]


TBLK = 32768


def _tc_transpose(wT):
    V = wT.shape[1]

    def body(in_ref, out_ref):
        r = jax.lax.broadcasted_iota(jnp.int32, (D, 2 * D), 0)
        c = jax.lax.broadcasted_iota(jnp.int32, (D, 2 * D), 1)
        eye = (r == c).astype(jnp.float32)
        out_ref[...] = jax.lax.dot_general(
            in_ref[...], eye, (((0,), (0,)), ((), ())),
            preferred_element_type=jnp.float32)

    return pl.pallas_call(
        body,
        grid=(pl.cdiv(V, TBLK),),
        in_specs=[pl.BlockSpec((D, TBLK), lambda i: (0, i))],
        out_specs=pl.BlockSpec((TBLK, 2 * D), lambda i: (i, 0)),
        out_shape=jax.ShapeDtypeStruct((V, 2 * D), jnp.float32),
    )(wT)


def kernel(pos_u, pos_v, neg_v, snd_u_weight, snd_v_weight):
    wu_row = _tc_transpose(snd_u_weight.T)
    wv_row = _tc_transpose(snd_v_weight.T)
    pos_s, neg_s = _sc_scores(pos_u, pos_v, neg_v.reshape(-1),
                              wu_row, wv_row)
    return _loss(pos_s, neg_s)
